# baseline (speedup 1.0000x reference)
import functools

import jax
import jax.numpy as jnp
from jax import lax
from jax.experimental import pallas as pl
from jax.experimental.pallas import tpu as pltpu
from jax.experimental.pallas import tpu_sc as plsc

B = 16384
D = 64
OUT = 64

_info = plsc.get_sparse_core_info()
NC = _info.num_cores
NS = _info.num_subcores
NW = NC * NS
BPW = B // NW
CHUNK = 256
NPH = BPW // CHUNK

_mesh = plsc.VectorSubcoreMesh(core_axis_name="c", subcore_axis_name="s")


@functools.partial(
    pl.kernel,
    mesh=_mesh,
    out_type=[
        jax.ShapeDtypeStruct((B, D), jnp.float32),
        jax.ShapeDtypeStruct((B, D), jnp.float32),
    ],
    scratch_types=[
        pltpu.VMEM((BPW,), jnp.int32),
        pltpu.VMEM((BPW,), jnp.int32),
        pltpu.VMEM((CHUNK, D), jnp.float32),
        pltpu.VMEM((CHUNK, D), jnp.float32),
        pltpu.SemaphoreType.DMA,
        pltpu.SemaphoreType.DMA,
        pltpu.SemaphoreType.DMA,
    ],
)
def _sc_gather(sidx_hbm, aidx_hbm, stable_hbm, atable_hbm, es_hbm, ea_hbm,
               sidx_v, aidx_v, sbuf, abuf, sgsem, agsem, wsem):
    wid = lax.axis_index("s") * NC + lax.axis_index("c")
    base = wid * BPW
    pltpu.sync_copy(sidx_hbm.at[wid], sidx_v)
    pltpu.sync_copy(aidx_hbm.at[wid], aidx_v)

    for p in range(NPH):
        def body(j, _, p=p):
            svec = sidx_v[pl.ds(p * CHUNK + j * 16, 16)]
            avec = aidx_v[pl.ds(p * CHUNK + j * 16, 16)]
            for k in range(16):
                pltpu.make_async_copy(
                    stable_hbm.at[pl.ds(svec[k], 1)],
                    sbuf.at[pl.ds(j * 16 + k, 1)], sgsem).start()
                pltpu.make_async_copy(
                    atable_hbm.at[pl.ds(avec[k], 1)],
                    abuf.at[pl.ds(j * 16 + k, 1)], agsem).start()
            return 0

        lax.fori_loop(0, CHUNK // 16, body, 0)
        if p > 0:
            pltpu.make_async_copy(
                sbuf, es_hbm.at[pl.ds(0, CHUNK)], wsem).wait()
            pltpu.make_async_copy(
                abuf, ea_hbm.at[pl.ds(0, CHUNK)], wsem).wait()
        pltpu.make_async_copy(
            stable_hbm.at[pl.ds(0, CHUNK)], sbuf, sgsem).wait()
        pltpu.make_async_copy(
            atable_hbm.at[pl.ds(0, CHUNK)], abuf, agsem).wait()
        dst = pl.ds(base + p * CHUNK, CHUNK)
        pltpu.make_async_copy(sbuf, es_hbm.at[dst], wsem).start()
        pltpu.make_async_copy(abuf, ea_hbm.at[dst], wsem).start()

    pltpu.make_async_copy(sbuf, es_hbm.at[pl.ds(0, CHUNK)], wsem).wait()
    pltpu.make_async_copy(abuf, ea_hbm.at[pl.ds(0, CHUNK)], wsem).wait()


BLK = 2048


def _mm_body(es_ref, ea_ref, w1_ref, w2_ref, b_ref, o_ref):
    o_ref[...] = (
        jnp.dot(es_ref[...], w1_ref[...], preferred_element_type=jnp.float32)
        + jnp.dot(ea_ref[...], w2_ref[...], preferred_element_type=jnp.float32)
        + b_ref[...]
    )


_mm = pl.pallas_call(
    _mm_body,
    grid=(B // BLK,),
    in_specs=[
        pl.BlockSpec((BLK, D), lambda i: (i, 0)),
        pl.BlockSpec((BLK, D), lambda i: (i, 0)),
        pl.BlockSpec((D, OUT), lambda i: (0, 0)),
        pl.BlockSpec((D, OUT), lambda i: (0, 0)),
        pl.BlockSpec((1, OUT), lambda i: (0, 0)),
    ],
    out_specs=pl.BlockSpec((BLK, OUT), lambda i: (i, 0)),
    out_shape=jax.ShapeDtypeStruct((B, OUT), jnp.float32),
)


def kernel(state, action, state_table, action_table, W, b):
    sidx = state.astype(jnp.int32).reshape(NW, BPW)
    aidx = action.astype(jnp.int32).reshape(NW, BPW)
    es, ea = _sc_gather(sidx, aidx, state_table, action_table)
    w1 = W[:, :D].T
    w2 = W[:, D:].T
    return _mm(es, ea, w1, w2, b.reshape(1, OUT))
